# baseline (speedup 1.0000x reference)
import functools
import math

import jax
import jax.numpy as jnp
from jax import lax
from jax.experimental import pallas as pl
from jax.experimental.pallas import tpu as pltpu
from jax.experimental.pallas import tpu_sc as plsc

_NC = 2
_NS = 16
_L = 16
_NW = _NC * _NS


def _sc_gather(neuron_regions, region_emb, r_map):
    B, N = neuron_regions.shape
    D = region_emb.shape[1]
    n_regions = r_map.shape[0]
    wpb = _NW // B
    per = N // wpb
    mesh = plsc.VectorSubcoreMesh(core_axis_name="c", subcore_axis_name="s")

    @functools.partial(
        pl.kernel,
        out_type=(jax.ShapeDtypeStruct((B, N, D), jnp.float32),
                  jax.ShapeDtypeStruct((B, N), jnp.int32)),
        mesh=mesh,
        scratch_types=[
            pltpu.VMEM((per,), jnp.int32),
            pltpu.VMEM((per, D), jnp.float32),
            pltpu.VMEM((n_regions,), jnp.int32),
            pltpu.VMEM((per,), jnp.int32),
            pltpu.SemaphoreType.DMA,
            pltpu.SemaphoreType.DMA,
            pltpu.SemaphoreType.DMA,
        ],
        compiler_params=pltpu.CompilerParams(needs_layout_passes=False),
    )
    def sc_body(nr_hbm, re_hbm, rmap_hbm, g_hbm, lr_hbm,
                idx_v, rows_v, rmap_v, lr_v, sem_a, sem_b, sem_c):
        wid = lax.axis_index("s") * _NC + lax.axis_index("c")
        for bb in range(B):
            @pl.when((wid >= bb * wpb) & (wid < (bb + 1) * wpb))
            def _():
                off = (wid - bb * wpb) * per
                cp_idx = pltpu.async_copy(nr_hbm.at[bb, pl.ds(off, per)],
                                          idx_v, sem_a)
                cp_rm = pltpu.async_copy(rmap_hbm, rmap_v, sem_b)
                cp_idx.wait()
                pltpu.async_copy(re_hbm.at[idx_v], rows_v, sem_c).wait()
                cp_g = pltpu.async_copy(rows_v, g_hbm.at[bb, pl.ds(off, per)],
                                        sem_a)
                cp_rm.wait()
                for i in range(per // _L):
                    idx = idx_v[pl.ds(i * _L, _L)]
                    lr_v[pl.ds(i * _L, _L)] = plsc.load_gather(rmap_v, [idx])
                pltpu.async_copy(lr_v, lr_hbm.at[bb, pl.ds(off, per)],
                                 sem_b).wait()
                cp_g.wait()

    return sc_body(neuron_regions, region_emb, r_map)


def _decoder_body(ut_ref, g_ref, lr_ref, eids_ref, ns_ref, ee_ref,
                  lng_ref, lnb_ref, w1_ref, b1_ref, w2_ref, b2_ref, out_ref):
    f32 = jnp.float32
    B, R, T, Ds = ut_ref.shape
    N = lr_ref.shape[1]
    n_eids = ee_ref.shape[0]
    d_id = ns_ref.shape[1]

    iota_r = lax.broadcasted_iota(jnp.int32, (R, N), 0)
    iota_eid = lax.broadcasted_iota(jnp.int32, (1, n_eids), 1)
    inv_sqrt2 = 1.0 / math.sqrt(2.0)
    lng = lng_ref[...].reshape(1, d_id)
    lnb = lnb_ref[...].reshape(1, d_id)
    b1r = b1_ref[...].reshape(1, -1)
    b2r = b2_ref[...].reshape(1, -1)

    for b in range(B):
        eid_oh = (eids_ref[b] == iota_eid).astype(f32)
        eid_row = jnp.dot(eid_oh, ee_ref[...], preferred_element_type=f32)

        e = ns_ref[...] + g_ref[b] + eid_row

        mu = jnp.mean(e, axis=1, keepdims=True)
        xc = e - mu
        var = jnp.mean(xc * xc, axis=1, keepdims=True)
        eh = xc * lax.rsqrt(var + 1e-5) * lng + lnb

        pre = jnp.dot(eh, w1_ref[...], preferred_element_type=f32) + b1r
        h = 0.5 * pre * (1.0 + lax.erf(pre * inv_sqrt2))
        wb = jnp.dot(h, w2_ref[...], preferred_element_type=f32) + b2r
        wbT = jnp.transpose(wb)
        wT = wbT[:Ds, :]
        biasT = wbT[Ds:Ds + 1, :]

        lr_row = lr_ref[pl.ds(b, 1), :]
        MT = (lr_row == iota_r).astype(f32)

        u_flat = ut_ref[b].reshape(R * T, Ds)
        pall = jnp.dot(u_flat, wT, preferred_element_type=f32)
        acc = biasT * jnp.ones((T, 1), f32)
        for r in range(R):
            acc = acc + pall[r * T:(r + 1) * T, :] * MT[r:r + 1, :]
        out_ref[b] = acc


def kernel(U, neuron_regions, eids, r_map, neuron_slot, region_emb, eid_emb,
           ln_g, ln_b, W1, b1, W2, b2):
    B, T, R, Ds = U.shape
    N = neuron_regions.shape[1]
    d_id = neuron_slot.shape[1]

    g, lr = _sc_gather(neuron_regions, region_emb, r_map)

    ut = U.transpose(0, 2, 1, 3)

    pred = pl.pallas_call(
        _decoder_body,
        out_shape=jax.ShapeDtypeStruct((B, T, N), jnp.float32),
        in_specs=[
            pl.BlockSpec(memory_space=pltpu.VMEM),
            pl.BlockSpec(memory_space=pltpu.VMEM),
            pl.BlockSpec(memory_space=pltpu.VMEM),
            pl.BlockSpec(memory_space=pltpu.SMEM),
            pl.BlockSpec(memory_space=pltpu.VMEM),
            pl.BlockSpec(memory_space=pltpu.VMEM),
            pl.BlockSpec(memory_space=pltpu.VMEM),
            pl.BlockSpec(memory_space=pltpu.VMEM),
            pl.BlockSpec(memory_space=pltpu.VMEM),
            pl.BlockSpec(memory_space=pltpu.VMEM),
            pl.BlockSpec(memory_space=pltpu.VMEM),
            pl.BlockSpec(memory_space=pltpu.VMEM),
        ],
        out_specs=pl.BlockSpec(memory_space=pltpu.VMEM),
    )(ut, g, lr, eids, neuron_slot[:N], eid_emb, ln_g, ln_b, W1, b1, W2, b2)
    return pred


# speedup vs baseline: 1.0940x; 1.0940x over previous
import functools
import math

import jax
import jax.numpy as jnp
from jax import lax
from jax.experimental import pallas as pl
from jax.experimental.pallas import tpu as pltpu
from jax.experimental.pallas import tpu_sc as plsc

_NC = 2
_NS = 16
_L = 16
_NW = _NC * _NS


def _sc_gather(neuron_regions, region_emb, r_map):
    B, N = neuron_regions.shape
    D = region_emb.shape[1]
    n_regions = r_map.shape[0]
    wpb = _NW // B
    per = N // wpb
    mesh = plsc.VectorSubcoreMesh(core_axis_name="c", subcore_axis_name="s")

    @functools.partial(
        pl.kernel,
        out_type=(jax.ShapeDtypeStruct((B, N, D), jnp.float32),
                  jax.ShapeDtypeStruct((B, 1, N), jnp.int32)),
        mesh=mesh,
        scratch_types=[
            pltpu.VMEM((per,), jnp.int32),
            pltpu.VMEM((per, D), jnp.float32),
            pltpu.VMEM((n_regions,), jnp.int32),
            pltpu.VMEM((per,), jnp.int32),
            pltpu.SemaphoreType.DMA,
            pltpu.SemaphoreType.DMA,
            pltpu.SemaphoreType.DMA,
        ],
        compiler_params=pltpu.CompilerParams(needs_layout_passes=False),
    )
    def sc_body(nr_hbm, re_hbm, rmap_hbm, g_hbm, lr_hbm,
                idx_v, rows_v, rmap_v, lr_v, sem_a, sem_b, sem_c):
        wid = lax.axis_index("s") * _NC + lax.axis_index("c")
        bb = wid // wpb
        off = (wid - bb * wpb) * per
        cp_idx = pltpu.async_copy(nr_hbm.at[bb, pl.ds(off, per)],
                                  idx_v, sem_a)
        cp_rm = pltpu.async_copy(rmap_hbm, rmap_v, sem_b)
        cp_idx.wait()
        pltpu.async_copy(re_hbm.at[idx_v], rows_v, sem_c).wait()
        cp_g = pltpu.async_copy(rows_v, g_hbm.at[bb, pl.ds(off, per)],
                                sem_a)
        cp_rm.wait()
        for i in range(per // _L):
            idx = idx_v[pl.ds(i * _L, _L)]
            lr_v[pl.ds(i * _L, _L)] = plsc.load_gather(rmap_v, [idx])
        pltpu.async_copy(lr_v, lr_hbm.at[bb, 0, pl.ds(off, per)],
                         sem_b).wait()
        cp_g.wait()

    return sc_body(neuron_regions, region_emb, r_map)


def _decoder_body(u_ref, g_ref, lr_ref, eids_ref, ns_ref, ee_ref,
                  lng_ref, lnb_ref, w1_ref, b1_ref, w2_ref, b2_ref, out_ref):
    f32 = jnp.float32
    _, T, R, Ds = u_ref.shape
    N = lr_ref.shape[2]
    n_eids = ee_ref.shape[0]
    d_id = ns_ref.shape[1]

    iota_r = lax.broadcasted_iota(jnp.int32, (R, N), 0)
    iota_eid = lax.broadcasted_iota(jnp.int32, (1, n_eids), 1)
    inv_sqrt2 = 1.0 / math.sqrt(2.0)
    lng = lng_ref[...].reshape(1, d_id)
    lnb = lnb_ref[...].reshape(1, d_id)
    b1r = b1_ref[...].reshape(1, -1)
    b2r = b2_ref[...].reshape(1, -1)

    b = pl.program_id(0)
    eid_oh = (eids_ref[b] == iota_eid).astype(f32)
    eid_row = jnp.dot(eid_oh, ee_ref[...], preferred_element_type=f32)

    e = ns_ref[...] + g_ref[0] + eid_row

    mu = jnp.mean(e, axis=1, keepdims=True)
    xc = e - mu
    var = jnp.mean(xc * xc, axis=1, keepdims=True)
    eh = xc * lax.rsqrt(var + 1e-5) * lng + lnb

    pre = jnp.dot(eh, w1_ref[...], preferred_element_type=f32) + b1r
    h = 0.5 * pre * (1.0 + lax.erf(pre * inv_sqrt2))
    wb = jnp.dot(h, w2_ref[...], preferred_element_type=f32) + b2r
    wbT = jnp.transpose(wb)
    wT = wbT[:Ds, :]
    biasT = wbT[Ds:Ds + 1, :]

    MT = (lr_ref[0] == iota_r).astype(f32)

    u_flat = u_ref[0].reshape(T * R, Ds)
    pall = jnp.dot(u_flat, wT, preferred_element_type=f32)
    pall3 = pall.reshape(T, R, N)
    acc = jnp.sum(pall3 * MT[None, :, :], axis=1)
    out_ref[0] = acc + biasT


def kernel(U, neuron_regions, eids, r_map, neuron_slot, region_emb, eid_emb,
           ln_g, ln_b, W1, b1, W2, b2):
    B, T, R, Ds = U.shape
    N = neuron_regions.shape[1]
    d_id = neuron_slot.shape[1]

    g, lr = _sc_gather(neuron_regions, region_emb, r_map)

    n_eids, _ = eid_emb.shape
    two_ds = W1.shape[1]
    pred = pl.pallas_call(
        _decoder_body,
        grid=(B,),
        out_shape=jax.ShapeDtypeStruct((B, T, N), jnp.float32),
        in_specs=[
            pl.BlockSpec((1, T, R, Ds), lambda b: (b, 0, 0, 0)),
            pl.BlockSpec((1, N, d_id), lambda b: (b, 0, 0)),
            pl.BlockSpec((1, 1, N), lambda b: (b, 0, 0)),
            pl.BlockSpec(memory_space=pltpu.SMEM),
            pl.BlockSpec((N, d_id), lambda b: (0, 0)),
            pl.BlockSpec((n_eids, d_id), lambda b: (0, 0)),
            pl.BlockSpec((d_id,), lambda b: (0,)),
            pl.BlockSpec((d_id,), lambda b: (0,)),
            pl.BlockSpec((d_id, two_ds), lambda b: (0, 0)),
            pl.BlockSpec((two_ds,), lambda b: (0,)),
            pl.BlockSpec((two_ds, Ds + 1), lambda b: (0, 0)),
            pl.BlockSpec((Ds + 1,), lambda b: (0,)),
        ],
        out_specs=pl.BlockSpec((1, T, N), lambda b: (b, 0, 0)),
    )(U, g, lr, eids, neuron_slot[:N], eid_emb, ln_g, ln_b, W1, b1, W2, b2)
    return pred
